# baseline (speedup 1.0000x reference)
import jax
import jax.numpy as jnp
from jax import lax
from jax.experimental import pallas as pl
from jax.experimental.pallas import tpu as pltpu
from jax.experimental.pallas import tpu_sc as plsc

R = 128
B = 16384
NC = 2
NS = 16
NW = NC * NS
L = 16
SPW = B // NW
C = 32
NCHUNK = SPW // C
NG = C // L
NU = 339
NSV = 5825
NSV_PAD = 5888
SROWS = NSV_PAD // NS
EPS = 1e-5


def _rsqrt(x):
    i = plsc.bitcast(x, jnp.int32)
    i = jnp.int32(0x5F3759DF) - (i >> 1)
    y = plsc.bitcast(i, jnp.float32)
    for _ in range(3):
        y = y * (1.5 - 0.5 * x * y * y)
    return y


def _sc_body(uidx_hbm, sidx_hbm, umapA_hbm, umapB_hbm,
             smapA_hbm, smapB_hbm, smapC_hbm,
             uemb_hbm, uas_hbm, ure_hbm,
             semb_hbm, sas_hbm, sre_hbm, spr_hbm,
             prm_hbm,
             out_hbm, ctab_hbm,
             uidx_v, sidx_v, umapA_v, umapB_v, smapA_v, smapB_v, smapC_v,
             ib0, ib1, ib2, ib3,
             utab_u, utab_as, utab_re,
             st0a, st0b, st0c, st0d, st1a, st1b, st1c, st1d, accb,
             prm_v, out_v, sem0, sem1, semi):
    cid = lax.axis_index("c")
    sid = lax.axis_index("s")
    wid = sid * NC + cid
    base = wid * SPW
    lanes = lax.iota(jnp.int32, L)
    sems = [sem0, sem1]
    stage = [(st0a, st0b, st0c, st0d), (st1a, st1b, st1c, st1d)]

    setup = [
        pltpu.async_copy(uidx_hbm.at[pl.ds(base, SPW)], uidx_v, semi),
        pltpu.async_copy(sidx_hbm.at[pl.ds(base, SPW)], sidx_v, semi),
        pltpu.async_copy(umapA_hbm, umapA_v, semi),
        pltpu.async_copy(umapB_hbm, umapB_v, semi),
        pltpu.async_copy(smapA_hbm, smapA_v, semi),
        pltpu.async_copy(smapB_hbm, smapB_v, semi),
        pltpu.async_copy(smapC_hbm, smapC_v, semi),
        pltpu.async_copy(uemb_hbm, utab_u, semi),
        pltpu.async_copy(uas_hbm, utab_as, semi),
        pltpu.async_copy(ure_hbm, utab_re, semi),
        pltpu.async_copy(prm_hbm, prm_v, semi),
    ]
    for d in setup:
        d.wait()

    NRG = (NU + L - 1) // L

    def ubuild(rg, col0):
        rv_raw = lanes + rg * L
        msk = rv_raw < NU
        rv = jnp.minimum(rv_raw, NU - 1)
        asr = plsc.load_gather(umapA_v, [rv])
        rer = plsc.load_gather(umapB_v, [rv])

        def cstep(f, col):
            val = (plsc.load_gather(utab_u, [rv, col])
                   + plsc.load_gather(utab_as, [asr, col])
                   + plsc.load_gather(utab_re, [rer, col]))
            plsc.store_scatter(utab_u, [rv, col], val, mask=msk)
            return (col + 1) & jnp.int32(127)
        return lax.fori_loop(0, R, cstep, col0)

    lax.fori_loop(0, NRG, ubuild, lanes)

    start0 = sid * SROWS
    cbase = cid * NSV_PAD

    def sderive_fire(k, s):
        rbase = start0 + k * C
        n = C if k < 11 else L
        for v in range(n // L):
            rv = jnp.minimum(rbase + v * L + lanes, NSV - 1)
            ib0[pl.ds(v * L, L)] = rv
            ib1[pl.ds(v * L, L)] = plsc.load_gather(smapA_v, [rv])
            ib2[pl.ds(v * L, L)] = plsc.load_gather(smapB_v, [rv])
            ib3[pl.ds(v * L, L)] = plsc.load_gather(smapC_v, [rv])
        st = stage[s]
        sm = sems[s]
        return [
            pltpu.async_copy(semb_hbm.at[ib0.at[pl.ds(0, n)]],
                             st[0].at[pl.ds(0, n)], sm),
            pltpu.async_copy(sas_hbm.at[ib1.at[pl.ds(0, n)]],
                             st[1].at[pl.ds(0, n)], sm),
            pltpu.async_copy(sre_hbm.at[ib2.at[pl.ds(0, n)]],
                             st[2].at[pl.ds(0, n)], sm),
            pltpu.async_copy(spr_hbm.at[ib3.at[pl.ds(0, n)]],
                             st[3].at[pl.ds(0, n)], sm),
        ]

    NKR = SROWS // C + 1
    bdescs = [None, None]
    bdescs[0] = sderive_fire(0, 0)
    wb = None
    for k in range(NKR):
        s = k % 2
        for d in bdescs[s]:
            d.wait()
        if k + 1 < NKR:
            bdescs[1 - s] = sderive_fire(k + 1, 1 - s)
        st = stage[s]
        n = C if k < 11 else L

        def comb(r, _):
            for j in range(R // L):
                sl = pl.ds(j * L, L)
                accb[r, sl] = (st[0][r, sl] + st[1][r, sl]
                               + st[2][r, sl] + st[3][r, sl])
            return 0
        lax.fori_loop(0, n, comb, 0)
        if wb is not None:
            wb.wait()
        wb = pltpu.async_copy(
            accb.at[pl.ds(0, n)],
            ctab_hbm.at[pl.ds(cbase + start0 + k * C, n)], semi)
    wb.wait()
    plsc.subcore_barrier()

    def _sum_param(k):
        acc = jnp.zeros((L,), jnp.float32)
        for j in range(R // L):
            acc = acc + prm_v[k, pl.ds(j * L, L)]
        return jnp.sum(acc)
    Sw = _sum_param(4)
    Sb = _sum_param(5)

    sbufs = [st0a, st1a]

    def derive_and_fire(c, s):
        lo = c * C
        for v in range(NG):
            sv = sidx_v[pl.ds(lo + v * L, L)]
            ib0[pl.ds(v * L, L)] = sv + cbase
        return [pltpu.async_copy(ctab_hbm.at[ib0], sbufs[s], sems[s])]

    inv_r = jnp.float32(1.0 / R)
    z = jnp.zeros((L,), jnp.float32)
    rows = [lanes + jnp.int32(g * L) for g in range(NG)]
    k_idx = [jnp.full((L,), k, jnp.int32) for k in range(5)]
    descs = [None, None]
    descs[0] = derive_and_fire(0, 0)

    for c in range(NCHUNK):
        s = c % 2
        for d in descs[s]:
            d.wait()
        if c + 1 < NCHUNK:
            descs[1 - s] = derive_and_fire(c + 1, 1 - s)
        bs = sbufs[s]

        lo = c * C
        uvr = [uidx_v[pl.ds(lo + g * L, L)] for g in range(NG)]

        def load_us(g, col):
            u = plsc.load_gather(utab_u, [uvr[g], col])
            sv = plsc.load_gather(bs, [rows[g], col])
            return u, sv

        init = tuple(((z, z, z, z)) for _ in range(NG)) + (lanes,)

        @plsc.parallel_loop(0, R, 1, unroll=4, carry=init)
        def res(i, acc):
            moms, col = acc[:-1], acc[-1]
            moms = list(moms)
            for g in range(NG):
                su, suu, ss, sss = moms[g]
                u, sv = load_us(g, col)
                moms[g] = (su + u, suu + u * u, ss + sv, sss + sv * sv)
            col = (col + 1) & jnp.int32(127)
            return tuple(moms) + (col,)

        stats = []
        for g in range(NG):
            su, suu, ss, sss = res[g]
            mu = su * inv_r
            ms = ss * inv_r
            iu = _rsqrt(suu * inv_r - mu * mu + EPS)
            isv = _rsqrt(sss * inv_r - ms * ms + EPS)
            stats.append((mu, ms, iu, isv))

        init2 = tuple(((z, z, z)) for _ in range(NG)) + (lanes,)

        @plsc.parallel_loop(0, R, 1, unroll=4, carry=init2)
        def res2(i, acc):
            moms, col = acc[:-1], acc[-1]
            moms = list(moms)
            uw = plsc.load_gather(prm_v, [k_idx[0], col])
            ub = plsc.load_gather(prm_v, [k_idx[1], col])
            sw = plsc.load_gather(prm_v, [k_idx[2], col])
            sb = plsc.load_gather(prm_v, [k_idx[3], col])
            w = plsc.load_gather(prm_v, [k_idx[4], col])
            for g in range(NG):
                mu, ms, iu, isv = stats[g]
                P, Q, W = moms[g]
                u, sv = load_us(g, col)
                un = (u - mu) * iu * uw + ub
                sn = (sv - ms) * isv * sw + sb
                prod = un * sn
                moms[g] = (P + prod, Q + prod * prod, W + prod * w)
            col = (col + 1) & jnp.int32(127)
            return tuple(moms) + (col,)

        for g in range(NG):
            P, Q, W = res2[g]
            m3 = P * inv_r
            i3 = _rsqrt(Q * inv_r - m3 * m3 + EPS)
            tmp = i3 * (W - m3 * Sw) + Sb
            pred = 1.0 / (1.0 + jnp.exp(-tmp))
            out_v[pl.ds(c * C + g * L, L)] = pred

    pltpu.async_copy(out_v, out_hbm.at[pl.ds(base, SPW)], semi).wait()


@jax.jit
def _csmf_sc(uidx, sidx, umapA, umapB, smapA, smapB, smapC,
             uemb, uas, ure, semb, sas, sre, spr, prm):
    mesh = plsc.VectorSubcoreMesh(core_axis_name="c", subcore_axis_name="s",
                                  num_cores=NC, num_subcores=NS)
    rowbuf = pltpu.VMEM((C, R), jnp.float32)
    idxbuf = pltpu.VMEM((C,), jnp.int32)
    f = pl.kernel(
        _sc_body,
        out_type=(jax.ShapeDtypeStruct((B,), jnp.float32),
                  jax.ShapeDtypeStruct((NC * NSV_PAD, R), jnp.float32)),
        mesh=mesh,
        compiler_params=pltpu.CompilerParams(needs_layout_passes=False),
        scratch_types=(
            [pltpu.VMEM((SPW,), jnp.int32)] * 2
            + [pltpu.VMEM((NU,), jnp.int32)] * 2
            + [pltpu.VMEM((NSV,), jnp.int32)] * 3
            + [idxbuf] * 4
            + [pltpu.VMEM((NU, R), jnp.float32),
               pltpu.VMEM((137, R), jnp.float32),
               pltpu.VMEM((31, R), jnp.float32)]
            + [rowbuf] * 9
            + [pltpu.VMEM((6, R), jnp.float32),
               pltpu.VMEM((SPW,), jnp.float32)]
            + [pltpu.SemaphoreType.DMA] * 3
        ),
    )
    return f(uidx, sidx, umapA, umapB, smapA, smapB, smapC,
             uemb, uas, ure, semb, sas, sre, spr, prm)[0]


def kernel(userIdx, servIdx, user_as_map, user_re_map, serv_as_map,
           serv_re_map, serv_pr_map, user_emb, uas_emb, ure_emb, serv_emb,
           sas_emb, sre_emb, spr_emb, user_ln_w, user_ln_b, serv_ln_w,
           serv_ln_b, norm_w, norm_b):
    prm = jnp.stack([user_ln_w, user_ln_b, serv_ln_w, serv_ln_b,
                     norm_w, norm_b]).astype(jnp.float32)
    return _csmf_sc(userIdx, servIdx, user_as_map, user_re_map, serv_as_map,
                    serv_re_map, serv_pr_map, user_emb, uas_emb, ure_emb,
                    serv_emb, sas_emb, sre_emb, spr_emb, prm)


# speedup vs baseline: 1.1441x; 1.1441x over previous
import jax
import jax.numpy as jnp
from jax import lax
from jax.experimental import pallas as pl
from jax.experimental.pallas import tpu as pltpu
from jax.experimental.pallas import tpu_sc as plsc

R = 128
B = 16384
NC = 2
NS = 16
NW = NC * NS
L = 16
SPW = B // NW
C = 32
NCHUNK = SPW // C
NG = C // L
NU = 339
NSV = 5825
NSV_PAD = 5888
SROWS = NSV_PAD // NS
EPS = 1e-5


def _rsqrt(x):
    i = plsc.bitcast(x, jnp.int32)
    i = jnp.int32(0x5F3759DF) - (i >> 1)
    y = plsc.bitcast(i, jnp.float32)
    for _ in range(3):
        y = y * (1.5 - 0.5 * x * y * y)
    return y


def _sc_body(uidx_hbm, sidx_hbm, umapA_hbm, umapB_hbm,
             smapA_hbm, smapB_hbm, smapC_hbm,
             uemb_hbm, uas_hbm, ure_hbm,
             semb_hbm, sas_hbm, sre_hbm, spr_hbm,
             prm_hbm,
             out_hbm, ctab_hbm,
             uidx_v, sidx_v, umapA_v, umapB_v, smapA_v, smapB_v, smapC_v,
             ib0, ib1, ib2, ib3,
             utab_u, utab_as, utab_re,
             st0a, st0b, st0c, st0d, st1a, st1b, st1c, st1d, accb,
             prm_v, out_v, sem0, sem1, semi):
    cid = lax.axis_index("c")
    sid = lax.axis_index("s")
    wid = sid * NC + cid
    base = wid * SPW
    lanes = lax.iota(jnp.int32, L)
    sems = [sem0, sem1]
    stage = [(st0a, st0b, st0c, st0d), (st1a, st1b, st1c, st1d)]

    setup = [
        pltpu.async_copy(uidx_hbm.at[pl.ds(base, SPW)], uidx_v, semi),
        pltpu.async_copy(sidx_hbm.at[pl.ds(base, SPW)], sidx_v, semi),
        pltpu.async_copy(umapA_hbm, umapA_v, semi),
        pltpu.async_copy(umapB_hbm, umapB_v, semi),
        pltpu.async_copy(smapA_hbm, smapA_v, semi),
        pltpu.async_copy(smapB_hbm, smapB_v, semi),
        pltpu.async_copy(smapC_hbm, smapC_v, semi),
        pltpu.async_copy(uemb_hbm, utab_u, semi),
        pltpu.async_copy(uas_hbm, utab_as, semi),
        pltpu.async_copy(ure_hbm, utab_re, semi),
        pltpu.async_copy(prm_hbm, prm_v, semi),
    ]
    for d in setup:
        d.wait()

    NRG = (NU + L - 1) // L

    def ubuild(rg, col0):
        rv_raw = lanes + rg * L
        msk = rv_raw < NU
        rv = jnp.minimum(rv_raw, NU - 1)
        asr = plsc.load_gather(umapA_v, [rv])
        rer = plsc.load_gather(umapB_v, [rv])

        @plsc.parallel_loop(0, R, 1, unroll=4, carry=col0)
        def colout(f, col):
            val = (plsc.load_gather(utab_u, [rv, col])
                   + plsc.load_gather(utab_as, [asr, col])
                   + plsc.load_gather(utab_re, [rer, col]))
            plsc.store_scatter(utab_u, [rv, col], val, mask=msk)
            return (col + 1) & jnp.int32(127)
        return colout

    lax.fori_loop(0, NRG, ubuild, lanes)

    start0 = sid * SROWS
    cbase = cid * NSV_PAD

    def sderive_fire(k, s):
        rbase = start0 + k * C
        n = C if k < 11 else L
        for v in range(n // L):
            rv = jnp.minimum(rbase + v * L + lanes, NSV - 1)
            ib0[pl.ds(v * L, L)] = rv
            ib1[pl.ds(v * L, L)] = plsc.load_gather(smapA_v, [rv])
            ib2[pl.ds(v * L, L)] = plsc.load_gather(smapB_v, [rv])
            ib3[pl.ds(v * L, L)] = plsc.load_gather(smapC_v, [rv])
        st = stage[s]
        sm = sems[s]
        return [
            pltpu.async_copy(semb_hbm.at[ib0.at[pl.ds(0, n)]],
                             st[0].at[pl.ds(0, n)], sm),
            pltpu.async_copy(sas_hbm.at[ib1.at[pl.ds(0, n)]],
                             st[1].at[pl.ds(0, n)], sm),
            pltpu.async_copy(sre_hbm.at[ib2.at[pl.ds(0, n)]],
                             st[2].at[pl.ds(0, n)], sm),
            pltpu.async_copy(spr_hbm.at[ib3.at[pl.ds(0, n)]],
                             st[3].at[pl.ds(0, n)], sm),
        ]

    NKR = SROWS // C + 1
    bdescs = [None, None]
    bdescs[0] = sderive_fire(0, 0)
    wb = None
    for k in range(NKR):
        s = k % 2
        for d in bdescs[s]:
            d.wait()
        if k + 1 < NKR:
            bdescs[1 - s] = sderive_fire(k + 1, 1 - s)
        st = stage[s]
        n = C if k < 11 else L

        @plsc.parallel_loop(0, n, 1, unroll=2)
        def comb(r):
            for j in range(R // L):
                sl = pl.ds(j * L, L)
                accb[r, sl] = (st[0][r, sl] + st[1][r, sl]
                               + st[2][r, sl] + st[3][r, sl])
        if wb is not None:
            wb.wait()
        wb = pltpu.async_copy(
            accb.at[pl.ds(0, n)],
            ctab_hbm.at[pl.ds(cbase + start0 + k * C, n)], semi)
    wb.wait()
    plsc.subcore_barrier()

    def _sum_param(k):
        acc = jnp.zeros((L,), jnp.float32)
        for j in range(R // L):
            acc = acc + prm_v[k, pl.ds(j * L, L)]
        return jnp.sum(acc)
    Sw = _sum_param(4)
    Sb = _sum_param(5)

    sbufs = [st0a, st1a]

    def derive_and_fire(c, s):
        lo = c * C
        for v in range(NG):
            sv = sidx_v[pl.ds(lo + v * L, L)]
            ib0[pl.ds(v * L, L)] = sv + cbase
        return [pltpu.async_copy(ctab_hbm.at[ib0], sbufs[s], sems[s])]

    inv_r = jnp.float32(1.0 / R)
    z = jnp.zeros((L,), jnp.float32)
    rows = [lanes + jnp.int32(g * L) for g in range(NG)]
    k_idx = [jnp.full((L,), k, jnp.int32) for k in range(5)]
    descs = [None, None]
    descs[0] = derive_and_fire(0, 0)

    for c in range(NCHUNK):
        s = c % 2
        for d in descs[s]:
            d.wait()
        if c + 1 < NCHUNK:
            descs[1 - s] = derive_and_fire(c + 1, 1 - s)
        bs = sbufs[s]

        lo = c * C
        uvr = [uidx_v[pl.ds(lo + g * L, L)] for g in range(NG)]

        def load_us(g, col):
            u = plsc.load_gather(utab_u, [uvr[g], col])
            sv = plsc.load_gather(bs, [rows[g], col])
            return u, sv

        init = tuple(((z, z, z, z)) for _ in range(NG)) + (lanes,)

        @plsc.parallel_loop(0, R, 1, unroll=4, carry=init)
        def res(i, acc):
            moms, col = acc[:-1], acc[-1]
            moms = list(moms)
            for g in range(NG):
                su, suu, ss, sss = moms[g]
                u, sv = load_us(g, col)
                moms[g] = (su + u, suu + u * u, ss + sv, sss + sv * sv)
            col = (col + 1) & jnp.int32(127)
            return tuple(moms) + (col,)

        stats = []
        for g in range(NG):
            su, suu, ss, sss = res[g]
            mu = su * inv_r
            ms = ss * inv_r
            iu = _rsqrt(suu * inv_r - mu * mu + EPS)
            isv = _rsqrt(sss * inv_r - ms * ms + EPS)
            stats.append((mu, ms, iu, isv))

        init2 = tuple(((z, z, z)) for _ in range(NG)) + (lanes,)

        @plsc.parallel_loop(0, R, 1, unroll=4, carry=init2)
        def res2(i, acc):
            moms, col = acc[:-1], acc[-1]
            moms = list(moms)
            uw = plsc.load_gather(prm_v, [k_idx[0], col])
            ub = plsc.load_gather(prm_v, [k_idx[1], col])
            sw = plsc.load_gather(prm_v, [k_idx[2], col])
            sb = plsc.load_gather(prm_v, [k_idx[3], col])
            w = plsc.load_gather(prm_v, [k_idx[4], col])
            for g in range(NG):
                mu, ms, iu, isv = stats[g]
                P, Q, W = moms[g]
                u, sv = load_us(g, col)
                un = (u - mu) * iu * uw + ub
                sn = (sv - ms) * isv * sw + sb
                prod = un * sn
                moms[g] = (P + prod, Q + prod * prod, W + prod * w)
            col = (col + 1) & jnp.int32(127)
            return tuple(moms) + (col,)

        for g in range(NG):
            P, Q, W = res2[g]
            m3 = P * inv_r
            i3 = _rsqrt(Q * inv_r - m3 * m3 + EPS)
            tmp = i3 * (W - m3 * Sw) + Sb
            pred = 1.0 / (1.0 + jnp.exp(-tmp))
            out_v[pl.ds(c * C + g * L, L)] = pred

    pltpu.async_copy(out_v, out_hbm.at[pl.ds(base, SPW)], semi).wait()


@jax.jit
def _csmf_sc(uidx, sidx, umapA, umapB, smapA, smapB, smapC,
             uemb, uas, ure, semb, sas, sre, spr, prm):
    mesh = plsc.VectorSubcoreMesh(core_axis_name="c", subcore_axis_name="s",
                                  num_cores=NC, num_subcores=NS)
    rowbuf = pltpu.VMEM((C, R), jnp.float32)
    idxbuf = pltpu.VMEM((C,), jnp.int32)
    f = pl.kernel(
        _sc_body,
        out_type=(jax.ShapeDtypeStruct((B,), jnp.float32),
                  jax.ShapeDtypeStruct((NC * NSV_PAD, R), jnp.float32)),
        mesh=mesh,
        compiler_params=pltpu.CompilerParams(needs_layout_passes=False),
        scratch_types=(
            [pltpu.VMEM((SPW,), jnp.int32)] * 2
            + [pltpu.VMEM((NU,), jnp.int32)] * 2
            + [pltpu.VMEM((NSV,), jnp.int32)] * 3
            + [idxbuf] * 4
            + [pltpu.VMEM((NU, R), jnp.float32),
               pltpu.VMEM((137, R), jnp.float32),
               pltpu.VMEM((31, R), jnp.float32)]
            + [rowbuf] * 9
            + [pltpu.VMEM((6, R), jnp.float32),
               pltpu.VMEM((SPW,), jnp.float32)]
            + [pltpu.SemaphoreType.DMA] * 3
        ),
    )
    return f(uidx, sidx, umapA, umapB, smapA, smapB, smapC,
             uemb, uas, ure, semb, sas, sre, spr, prm)[0]


def kernel(userIdx, servIdx, user_as_map, user_re_map, serv_as_map,
           serv_re_map, serv_pr_map, user_emb, uas_emb, ure_emb, serv_emb,
           sas_emb, sre_emb, spr_emb, user_ln_w, user_ln_b, serv_ln_w,
           serv_ln_b, norm_w, norm_b):
    prm = jnp.stack([user_ln_w, user_ln_b, serv_ln_w, serv_ln_b,
                     norm_w, norm_b]).astype(jnp.float32)
    return _csmf_sc(userIdx, servIdx, user_as_map, user_re_map, serv_as_map,
                    serv_re_map, serv_pr_map, user_emb, uas_emb, ure_emb,
                    serv_emb, sas_emb, sre_emb, spr_emb, prm)


# speedup vs baseline: 1.3006x; 1.1368x over previous
import jax
import jax.numpy as jnp
from jax import lax
from jax.experimental import pallas as pl
from jax.experimental.pallas import tpu as pltpu
from jax.experimental.pallas import tpu_sc as plsc

R = 128
B = 16384
NC = 2
NS = 16
NW = NC * NS
L = 16
SPW = B // NW
C = 32
NCHUNK = SPW // C
NG = C // L
EPS = 1e-5


def _rsqrt(x):
    i = plsc.bitcast(x, jnp.int32)
    i = jnp.int32(0x5F3759DF) - (i >> 1)
    y = plsc.bitcast(i, jnp.float32)
    for _ in range(3):
        y = y * (1.5 - 0.5 * x * y * y)
    return y


def _sc_body(uidx_hbm, sidx_hbm, umapA_hbm, umapB_hbm,
             smapA_hbm, smapB_hbm, smapC_hbm,
             uemb_hbm, uas_hbm, ure_hbm,
             semb_hbm, sas_hbm, sre_hbm, spr_hbm,
             prm_hbm,
             out_hbm,
             uidx_v, sidx_v, umapA_v, umapB_v, smapA_v, smapB_v, smapC_v,
             sasi_v, srei_v, spri_v,
             utab_u, utab_as, utab_re,
             bs0, bsas0, bsre0, bspr0,
             bs1, bsas1, bsre1, bspr1,
             prm_v, out_v, sem0, sem1, semi):
    wid = lax.axis_index("s") * NC + lax.axis_index("c")
    base = wid * SPW
    bufs = [(bs0, bsas0, bsre0, bspr0), (bs1, bsas1, bsre1, bspr1)]
    sems = [sem0, sem1]

    setup = [
        pltpu.async_copy(uidx_hbm.at[pl.ds(base, SPW)], uidx_v, semi),
        pltpu.async_copy(sidx_hbm.at[pl.ds(base, SPW)], sidx_v, semi),
        pltpu.async_copy(umapA_hbm, umapA_v, semi),
        pltpu.async_copy(umapB_hbm, umapB_v, semi),
        pltpu.async_copy(smapA_hbm, smapA_v, semi),
        pltpu.async_copy(smapB_hbm, smapB_v, semi),
        pltpu.async_copy(smapC_hbm, smapC_v, semi),
        pltpu.async_copy(uemb_hbm, utab_u, semi),
        pltpu.async_copy(uas_hbm, utab_as, semi),
        pltpu.async_copy(ure_hbm, utab_re, semi),
        pltpu.async_copy(prm_hbm, prm_v, semi),
    ]
    for d in setup:
        d.wait()

    def _sum_param(k):
        acc = jnp.zeros((L,), jnp.float32)
        for j in range(R // L):
            acc = acc + prm_v[k, pl.ds(j * L, L)]
        return jnp.sum(acc)
    Sw = _sum_param(4)
    Sb = _sum_param(5)

    def derive_and_fire(c, s):
        lo = c * C
        for v in range(NG):
            sv = sidx_v[pl.ds(lo + v * L, L)]
            sasi_v[pl.ds(v * L, L)] = plsc.load_gather(smapA_v, [sv])
            srei_v[pl.ds(v * L, L)] = plsc.load_gather(smapB_v, [sv])
            spri_v[pl.ds(v * L, L)] = plsc.load_gather(smapC_v, [sv])
        b = bufs[s]
        sm = sems[s]
        return [
            pltpu.async_copy(semb_hbm.at[sidx_v.at[pl.ds(lo, C)]], b[0], sm),
            pltpu.async_copy(sas_hbm.at[sasi_v], b[1], sm),
            pltpu.async_copy(sre_hbm.at[srei_v], b[2], sm),
            pltpu.async_copy(spr_hbm.at[spri_v], b[3], sm),
        ]

    inv_r = jnp.float32(1.0 / R)
    z = jnp.zeros((L,), jnp.float32)
    lanes = lax.iota(jnp.int32, L)
    rows = [lanes + jnp.int32(g * L) for g in range(NG)]
    k_idx = [jnp.full((L,), k, jnp.int32) for k in range(5)]
    descs = [None, None]
    descs[0] = derive_and_fire(0, 0)

    for c in range(NCHUNK):
        s = c % 2
        for d in descs[s]:
            d.wait()
        if c + 1 < NCHUNK:
            descs[1 - s] = derive_and_fire(c + 1, 1 - s)
        b = bufs[s]

        lo = c * C
        uvr, uasr, urer = [], [], []
        for g in range(NG):
            uv = uidx_v[pl.ds(lo + g * L, L)]
            uvr.append(uv)
            uasr.append(plsc.load_gather(umapA_v, [uv]))
            urer.append(plsc.load_gather(umapB_v, [uv]))

        def load_us(g, col):
            u = (plsc.load_gather(utab_u, [uvr[g], col])
                 + plsc.load_gather(utab_as, [uasr[g], col])
                 + plsc.load_gather(utab_re, [urer[g], col]))
            sv = (plsc.load_gather(b[0], [rows[g], col])
                  + plsc.load_gather(b[1], [rows[g], col])
                  + plsc.load_gather(b[2], [rows[g], col])
                  + plsc.load_gather(b[3], [rows[g], col]))
            return u, sv

        init = tuple(((z, z, z, z)) for _ in range(NG)) + (lanes,)

        @plsc.parallel_loop(0, R, 1, unroll=8, carry=init)
        def res(i, acc):
            moms, col = acc[:-1], acc[-1]
            moms = list(moms)
            for g in range(NG):
                su, suu, ss, sss = moms[g]
                u, sv = load_us(g, col)
                moms[g] = (su + u, suu + u * u, ss + sv, sss + sv * sv)
            col = (col + 1) & jnp.int32(127)
            return tuple(moms) + (col,)

        stats = []
        for g in range(NG):
            su, suu, ss, sss = res[g]
            mu = su * inv_r
            ms = ss * inv_r
            iu = _rsqrt(suu * inv_r - mu * mu + EPS)
            isv = _rsqrt(sss * inv_r - ms * ms + EPS)
            stats.append((mu, ms, iu, isv))

        init2 = tuple(((z, z, z)) for _ in range(NG)) + (lanes,)

        @plsc.parallel_loop(0, R, 1, unroll=8, carry=init2)
        def res2(i, acc):
            moms, col = acc[:-1], acc[-1]
            moms = list(moms)
            uw = plsc.load_gather(prm_v, [k_idx[0], col])
            ub = plsc.load_gather(prm_v, [k_idx[1], col])
            sw = plsc.load_gather(prm_v, [k_idx[2], col])
            sb = plsc.load_gather(prm_v, [k_idx[3], col])
            w = plsc.load_gather(prm_v, [k_idx[4], col])
            for g in range(NG):
                mu, ms, iu, isv = stats[g]
                P, Q, W = moms[g]
                u, sv = load_us(g, col)
                un = (u - mu) * iu * uw + ub
                sn = (sv - ms) * isv * sw + sb
                prod = un * sn
                moms[g] = (P + prod, Q + prod * prod, W + prod * w)
            col = (col + 1) & jnp.int32(127)
            return tuple(moms) + (col,)

        for g in range(NG):
            P, Q, W = res2[g]
            m3 = P * inv_r
            i3 = _rsqrt(Q * inv_r - m3 * m3 + EPS)
            tmp = i3 * (W - m3 * Sw) + Sb
            pred = 1.0 / (1.0 + jnp.exp(-tmp))
            out_v[pl.ds(c * C + g * L, L)] = pred

    pltpu.async_copy(out_v, out_hbm.at[pl.ds(base, SPW)], semi).wait()


@jax.jit
def _csmf_sc(uidx, sidx, umapA, umapB, smapA, smapB, smapC,
             uemb, uas, ure, semb, sas, sre, spr, prm):
    mesh = plsc.VectorSubcoreMesh(core_axis_name="c", subcore_axis_name="s",
                                  num_cores=NC, num_subcores=NS)
    rowbuf = pltpu.VMEM((C, R), jnp.float32)
    idxbuf = pltpu.VMEM((C,), jnp.int32)
    f = pl.kernel(
        _sc_body,
        out_type=jax.ShapeDtypeStruct((B,), jnp.float32),
        mesh=mesh,
        compiler_params=pltpu.CompilerParams(needs_layout_passes=False),
        scratch_types=(
            [pltpu.VMEM((SPW,), jnp.int32)] * 2
            + [pltpu.VMEM((339,), jnp.int32)] * 2
            + [pltpu.VMEM((5825,), jnp.int32)] * 3
            + [idxbuf] * 3
            + [pltpu.VMEM((339, R), jnp.float32),
               pltpu.VMEM((137, R), jnp.float32),
               pltpu.VMEM((31, R), jnp.float32)]
            + [rowbuf] * 8
            + [pltpu.VMEM((6, R), jnp.float32),
               pltpu.VMEM((SPW,), jnp.float32)]
            + [pltpu.SemaphoreType.DMA] * 3
        ),
    )
    return f(uidx, sidx, umapA, umapB, smapA, smapB, smapC,
             uemb, uas, ure, semb, sas, sre, spr, prm)


def kernel(userIdx, servIdx, user_as_map, user_re_map, serv_as_map,
           serv_re_map, serv_pr_map, user_emb, uas_emb, ure_emb, serv_emb,
           sas_emb, sre_emb, spr_emb, user_ln_w, user_ln_b, serv_ln_w,
           serv_ln_b, norm_w, norm_b):
    prm = jnp.stack([user_ln_w, user_ln_b, serv_ln_w, serv_ln_b,
                     norm_w, norm_b]).astype(jnp.float32)
    return _csmf_sc(userIdx, servIdx, user_as_map, user_re_map, serv_as_map,
                    serv_re_map, serv_pr_map, user_emb, uas_emb, ure_emb,
                    serv_emb, sas_emb, sre_emb, spr_emb, prm)


# speedup vs baseline: 1.3198x; 1.0148x over previous
import jax
import jax.numpy as jnp
from jax import lax
from jax.experimental import pallas as pl
from jax.experimental.pallas import tpu as pltpu
from jax.experimental.pallas import tpu_sc as plsc

R = 128
B = 16384
NC = 2
NS = 16
NW = NC * NS
L = 16
SPW = B // NW
C = 32
NCHUNK = SPW // C
NG = C // L
EPS = 1e-5


def _rsqrt(x):
    i = plsc.bitcast(x, jnp.int32)
    i = jnp.int32(0x5F3759DF) - (i >> 1)
    y = plsc.bitcast(i, jnp.float32)
    for _ in range(3):
        y = y * (1.5 - 0.5 * x * y * y)
    return y


def _sc_body(uidx_hbm, sidx_hbm, umapA_hbm, umapB_hbm,
             smapA_hbm, smapB_hbm, smapC_hbm,
             uemb_hbm, uas_hbm, ure_hbm,
             semb_hbm, sas_hbm, sre_hbm, spr_hbm,
             prm_hbm,
             out_hbm,
             uidx_v, sidx_v, umapA_v, umapB_v, smapA_v, smapB_v, smapC_v,
             sasi_v, srei_v, spri_v,
             utab_u, utab_as, utab_re,
             bs0, bsas0, bsre0, bspr0,
             bs1, bsas1, bsre1, bspr1,
             prm_v, out_v, sem0, sem1, semi):
    wid = lax.axis_index("s") * NC + lax.axis_index("c")
    base = wid * SPW
    bufs = [(bs0, bsas0, bsre0, bspr0), (bs1, bsas1, bsre1, bspr1)]
    sems = [sem0, sem1]

    setup = [
        pltpu.async_copy(uidx_hbm.at[pl.ds(base, SPW)], uidx_v, semi),
        pltpu.async_copy(sidx_hbm.at[pl.ds(base, SPW)], sidx_v, semi),
        pltpu.async_copy(umapA_hbm, umapA_v, semi),
        pltpu.async_copy(umapB_hbm, umapB_v, semi),
        pltpu.async_copy(smapA_hbm, smapA_v, semi),
        pltpu.async_copy(smapB_hbm, smapB_v, semi),
        pltpu.async_copy(smapC_hbm, smapC_v, semi),
        pltpu.async_copy(uemb_hbm, utab_u, semi),
        pltpu.async_copy(uas_hbm, utab_as, semi),
        pltpu.async_copy(ure_hbm, utab_re, semi),
        pltpu.async_copy(prm_hbm, prm_v, semi),
    ]
    for d in setup:
        d.wait()

    def _sum_param(k):
        acc = jnp.zeros((L,), jnp.float32)
        for j in range(R // L):
            acc = acc + prm_v[k, pl.ds(j * L, L)]
        return jnp.sum(acc)
    Sw = _sum_param(4)
    Sb = _sum_param(5)

    def derive_and_fire(c, s):
        lo = c * C
        for v in range(NG):
            sv = sidx_v[pl.ds(lo + v * L, L)]
            sasi_v[pl.ds(v * L, L)] = plsc.load_gather(smapA_v, [sv])
            srei_v[pl.ds(v * L, L)] = plsc.load_gather(smapB_v, [sv])
            spri_v[pl.ds(v * L, L)] = plsc.load_gather(smapC_v, [sv])
        b = bufs[s]
        sm = sems[s]
        return [
            pltpu.async_copy(semb_hbm.at[sidx_v.at[pl.ds(lo, C)]], b[0], sm),
            pltpu.async_copy(sas_hbm.at[sasi_v], b[1], sm),
            pltpu.async_copy(sre_hbm.at[srei_v], b[2], sm),
            pltpu.async_copy(spr_hbm.at[spri_v], b[3], sm),
        ]

    inv_r = jnp.float32(1.0 / R)
    z = jnp.zeros((L,), jnp.float32)
    lanes = lax.iota(jnp.int32, L)
    rows = [lanes + jnp.int32(g * L) for g in range(NG)]
    k_idx = [jnp.full((L,), k, jnp.int32) for k in range(5)]
    descs = [None, None]
    descs[0] = derive_and_fire(0, 0)

    for c in range(NCHUNK):
        s = c % 2
        for d in descs[s]:
            d.wait()
        if c + 1 < NCHUNK:
            descs[1 - s] = derive_and_fire(c + 1, 1 - s)
        b = bufs[s]

        lo = c * C
        uvr, uasr, urer = [], [], []
        for g in range(NG):
            uv = uidx_v[pl.ds(lo + g * L, L)]
            uvr.append(uv)
            uasr.append(plsc.load_gather(umapA_v, [uv]))
            urer.append(plsc.load_gather(umapB_v, [uv]))

        def load_us(g, col):
            u = (plsc.load_gather(utab_u, [uvr[g], col])
                 + plsc.load_gather(utab_as, [uasr[g], col])
                 + plsc.load_gather(utab_re, [urer[g], col]))
            sv = (plsc.load_gather(b[0], [rows[g], col])
                  + plsc.load_gather(b[1], [rows[g], col])
                  + plsc.load_gather(b[2], [rows[g], col])
                  + plsc.load_gather(b[3], [rows[g], col]))
            return u, sv

        init = tuple(((z, z, z, z)) for _ in range(NG)) + (lanes,)

        @plsc.parallel_loop(0, R, 1, unroll=4, carry=init)
        def res(i, acc):
            moms, col = acc[:-1], acc[-1]
            moms = list(moms)
            for g in range(NG):
                su, suu, ss, sss = moms[g]
                u, sv = load_us(g, col)
                moms[g] = (su + u, suu + u * u, ss + sv, sss + sv * sv)
            col = (col + 1) & jnp.int32(127)
            return tuple(moms) + (col,)

        stats = []
        for g in range(NG):
            su, suu, ss, sss = res[g]
            mu = su * inv_r
            ms = ss * inv_r
            iu = _rsqrt(suu * inv_r - mu * mu + EPS)
            isv = _rsqrt(sss * inv_r - ms * ms + EPS)
            stats.append((mu, ms, iu, isv))

        init2 = tuple(((z, z, z)) for _ in range(NG)) + (lanes,)

        @plsc.parallel_loop(0, R, 1, unroll=4, carry=init2)
        def res2(i, acc):
            moms, col = acc[:-1], acc[-1]
            moms = list(moms)
            uw = plsc.load_gather(prm_v, [k_idx[0], col])
            ub = plsc.load_gather(prm_v, [k_idx[1], col])
            sw = plsc.load_gather(prm_v, [k_idx[2], col])
            sb = plsc.load_gather(prm_v, [k_idx[3], col])
            w = plsc.load_gather(prm_v, [k_idx[4], col])
            for g in range(NG):
                mu, ms, iu, isv = stats[g]
                P, Q, W = moms[g]
                u, sv = load_us(g, col)
                un = (u - mu) * iu * uw + ub
                sn = (sv - ms) * isv * sw + sb
                prod = un * sn
                moms[g] = (P + prod, Q + prod * prod, W + prod * w)
            col = (col + 1) & jnp.int32(127)
            return tuple(moms) + (col,)

        for g in range(NG):
            P, Q, W = res2[g]
            m3 = P * inv_r
            i3 = _rsqrt(Q * inv_r - m3 * m3 + EPS)
            tmp = i3 * (W - m3 * Sw) + Sb
            pred = 1.0 / (1.0 + jnp.exp(-tmp))
            out_v[pl.ds(c * C + g * L, L)] = pred

    pltpu.async_copy(out_v, out_hbm.at[pl.ds(base, SPW)], semi).wait()


@jax.jit
def _csmf_sc(uidx, sidx, umapA, umapB, smapA, smapB, smapC,
             uemb, uas, ure, semb, sas, sre, spr, prm):
    mesh = plsc.VectorSubcoreMesh(core_axis_name="c", subcore_axis_name="s",
                                  num_cores=NC, num_subcores=NS)
    rowbuf = pltpu.VMEM((C, R), jnp.float32)
    idxbuf = pltpu.VMEM((C,), jnp.int32)
    f = pl.kernel(
        _sc_body,
        out_type=jax.ShapeDtypeStruct((B,), jnp.float32),
        mesh=mesh,
        compiler_params=pltpu.CompilerParams(needs_layout_passes=False),
        scratch_types=(
            [pltpu.VMEM((SPW,), jnp.int32)] * 2
            + [pltpu.VMEM((339,), jnp.int32)] * 2
            + [pltpu.VMEM((5825,), jnp.int32)] * 3
            + [idxbuf] * 3
            + [pltpu.VMEM((339, R), jnp.float32),
               pltpu.VMEM((137, R), jnp.float32),
               pltpu.VMEM((31, R), jnp.float32)]
            + [rowbuf] * 8
            + [pltpu.VMEM((6, R), jnp.float32),
               pltpu.VMEM((SPW,), jnp.float32)]
            + [pltpu.SemaphoreType.DMA] * 3
        ),
    )
    return f(uidx, sidx, umapA, umapB, smapA, smapB, smapC,
             uemb, uas, ure, semb, sas, sre, spr, prm)


def kernel(userIdx, servIdx, user_as_map, user_re_map, serv_as_map,
           serv_re_map, serv_pr_map, user_emb, uas_emb, ure_emb, serv_emb,
           sas_emb, sre_emb, spr_emb, user_ln_w, user_ln_b, serv_ln_w,
           serv_ln_b, norm_w, norm_b):
    prm = jnp.stack([user_ln_w, user_ln_b, serv_ln_w, serv_ln_b,
                     norm_w, norm_b]).astype(jnp.float32)
    return _csmf_sc(userIdx, servIdx, user_as_map, user_re_map, serv_as_map,
                    serv_re_map, serv_pr_map, user_emb, uas_emb, ure_emb,
                    serv_emb, sas_emb, sre_emb, spr_emb, prm)
